# Initial kernel scaffold; baseline (speedup 1.0000x reference)
import jax
import jax.numpy as jnp
from jax.experimental import pallas as pl


def kernel(input_ids, emb_table, glove_table):
    raise NotImplementedError("write your pallas kernel here")


# speedup vs baseline: 3.4046x; 3.4046x over previous
import functools

import jax
import jax.numpy as jnp
from jax import lax
from jax.experimental import pallas as pl
from jax.experimental.pallas import tpu as pltpu
from jax.experimental.pallas import tpu_sc as plsc

_VOCAB = 100000
_D = 128
_DD = 2 * _D
_B = 4096
_L = 50
_BL = _B * _L

_info = plsc.get_sparse_core_info()
_NC, _NS = _info.num_cores, _info.num_subcores
_NW = _NC * _NS
_PER_W = _BL // _NW
_C = 128
_N_CH = _PER_W // _C


def _fuse_body(emb_ref, glove_ref, out_ref):
    out_ref[:, :_D] = jnp.tanh(emb_ref[:])
    out_ref[:, _D:] = glove_ref[:]


def _fuse_tables(emb, glove):
    blk = 2000
    return pl.pallas_call(
        _fuse_body,
        grid=(_VOCAB // blk,),
        in_specs=[
            pl.BlockSpec((blk, _D), lambda i: (i, 0)),
            pl.BlockSpec((blk, _D), lambda i: (i, 0)),
        ],
        out_specs=pl.BlockSpec((blk, _DD), lambda i: (i, 0)),
        out_shape=jax.ShapeDtypeStruct((_VOCAB, _DD), jnp.float32),
    )(emb, glove)


_mesh = plsc.VectorSubcoreMesh(core_axis_name="c", subcore_axis_name="s")


@functools.partial(
    pl.kernel,
    out_type=jax.ShapeDtypeStruct((_BL, _DD), jnp.float32),
    mesh=_mesh,
    scratch_types=[
        pltpu.VMEM((_C,), jnp.int32),
        pltpu.VMEM((_C,), jnp.int32),
        pltpu.VMEM((_C, _DD), jnp.float32),
        pltpu.VMEM((_C, _DD), jnp.float32),
        pltpu.SemaphoreType.DMA,
        pltpu.SemaphoreType.DMA,
    ],
)
def _sc_gather(tbl, ids, out, idx0, idx1, rows0, rows1, sem0, sem1):
    wid = lax.axis_index("s") * _NC + lax.axis_index("c")
    base = wid * _PER_W

    idx = (idx0, idx1)
    rows = (rows0, rows1)
    sem = (sem0, sem1)

    def start(buf, c):
        pltpu.sync_copy(ids.at[pl.ds(base + c * _C, _C)], idx[buf])
        pltpu.async_copy(tbl.at[idx[buf]], rows[buf], sem[buf])

    def drain(buf, c):
        pltpu.make_async_copy(tbl.at[idx[buf]], rows[buf], sem[buf]).wait()
        pltpu.sync_copy(rows[buf], out.at[pl.ds(base + c * _C, _C)])

    start(0, 0)

    def pair(g, carry):
        c = 2 * g
        start(1, c + 1)
        drain(0, c)

        @pl.when(c + 2 < _N_CH)
        def _():
            start(0, c + 2)

        drain(1, c + 1)
        return carry

    lax.fori_loop(0, _N_CH // 2, pair, 0)


def kernel(input_ids, emb_table, glove_table):
    ids = input_ids.reshape(-1).astype(jnp.int32)
    tbl = _fuse_tables(emb_table, glove_table)
    out = _sc_gather(tbl, ids)
    return out.reshape(input_ids.shape[0], input_ids.shape[1], _DD)
